# baseline (speedup 1.0000x reference)
import functools

import jax
import jax.numpy as jnp
from jax import lax
from jax.experimental import pallas as pl
from jax.experimental.pallas import tpu as pltpu

_NCHUNK = 64


def _sweep_kernel(ru, rv, su_ref, pu_ref, sv_ref, pv_ref, stu_ref, stv_ref,
                  ut_chunk, it_chunk,
                  out_ref):
    c = pl.program_id(0)
    g = pl.program_id(1)

    @pl.when(c == 0)
    def _():
        base = g * ru

        def body(i, _):
            out_ref[pu_ref[i]] = ut_chunk[su_ref[i] - base]
            return 0

        lax.fori_loop(stu_ref[g], stu_ref[g + 1], body, 0)

    @pl.when(c == 1)
    def _():
        base = g * rv

        def body(i, _):
            out_ref[pv_ref[i]] = it_chunk[sv_ref[i] - base]
            return 0

        lax.fori_loop(stv_ref[g], stv_ref[g + 1], body, 0)


def _dot_kernel(u_ref, v_ref, o_ref):
    w = u_ref[:, 0, :] * v_ref[:, 0, :]
    o_ref[...] = jnp.sum(w, axis=1, keepdims=True)


def kernel(u, v, user_tab, item_tab):
    B = u.shape[0]
    nu, ep = user_tab.shape
    ni = item_tab.shape[0]
    ru = nu // _NCHUNK
    rv = ni // _NCHUNK

    u32 = u.astype(jnp.int32).reshape(B)
    v32 = v.astype(jnp.int32).reshape(B)
    iota = lax.iota(jnp.int32, B)
    su, pu = lax.sort_key_val(u32, iota)
    sv, pv = lax.sort_key_val(v32, iota)
    stu = jnp.searchsorted(su, lax.iota(jnp.int32, _NCHUNK + 1) * ru
                           ).astype(jnp.int32)
    stv = jnp.searchsorted(sv, lax.iota(jnp.int32, _NCHUNK + 1) * rv
                           ).astype(jnp.int32)

    ut3 = user_tab.reshape(nu, 1, ep)
    it3 = item_tab.reshape(ni, 1, ep)

    grid_spec = pltpu.PrefetchScalarGridSpec(
        num_scalar_prefetch=6,
        grid=(2, _NCHUNK),
        in_specs=[
            pl.BlockSpec((ru, 1, ep),
                         lambda c, g, *_: (jnp.where(c == 0, g, 0), 0, 0)),
            pl.BlockSpec((rv, 1, ep),
                         lambda c, g, *_: (jnp.where(c == 1, g, 0), 0, 0)),
        ],
        out_specs=pl.BlockSpec((B, 1, ep), lambda c, g, *_: (c, 0, 0)),
    )
    rows = pl.pallas_call(
        functools.partial(_sweep_kernel, ru, rv),
        out_shape=jax.ShapeDtypeStruct((2 * B, 1, ep), jnp.float32),
        grid_spec=grid_spec,
        compiler_params=pltpu.CompilerParams(
            dimension_semantics=("parallel", "arbitrary"),
            disable_bounds_checks=True),
    )(su, pu, sv, pv, stu, stv, ut3, it3)

    blk = 1024
    nblk = B // blk
    out = pl.pallas_call(
        _dot_kernel,
        out_shape=jax.ShapeDtypeStruct((B, 1), jnp.float32),
        grid=(nblk,),
        in_specs=[
            pl.BlockSpec((blk, 1, ep), lambda i: (i, 0, 0)),
            pl.BlockSpec((blk, 1, ep), lambda i: (i + nblk, 0, 0)),
        ],
        out_specs=pl.BlockSpec((blk, 1), lambda i: (i, 0)),
        compiler_params=pltpu.CompilerParams(
            dimension_semantics=("parallel",),
            disable_bounds_checks=True),
    )(rows, rows)
    return out[:, 0]
